# baseline (speedup 1.0000x reference)
import functools

import jax
import jax.numpy as jnp
import numpy as np
from jax import lax
from jax.experimental import pallas as pl
from jax.experimental.pallas import tpu as pltpu
from jax.experimental.pallas import tpu_sc as plsc

E = 64
K = 8
G = 8
KG = 3
GS = E // G
NEG = -3.0e38
L = 16


def _matmul_kernel(h_ref, w_ref, out_ref):
    out_ref[...] = jax.lax.dot_general(
        h_ref[...], w_ref[...], (((1,), (1,)), ((), ())),
        preferred_element_type=jnp.float32)


def _tc_logits(hidden_states, weight):
    T, D = hidden_states.shape
    B = 2048
    return pl.pallas_call(
        _matmul_kernel,
        grid=(T // B,),
        in_specs=[
            pl.BlockSpec((B, D), lambda i: (i, 0)),
            pl.BlockSpec((E, D), lambda i: (0, 0)),
        ],
        out_specs=pl.BlockSpec((B, E), lambda i: (i, 0)),
        out_shape=jax.ShapeDtypeStruct((T, E), jnp.float32),
    )(hidden_states, weight)


def _vgather(x, idx):
    return lax.gather(
        x, idx[:, None],
        dimension_numbers=lax.GatherDimensionNumbers(
            offset_dims=(), collapsed_slice_dims=(0,), start_index_map=(0,)),
        slice_sizes=(1,),
        mode=lax.GatherScatterMode.PROMISE_IN_BOUNDS)


def _const_i32(v):
    return jnp.asarray(np.asarray(v, np.int32))


def _const_f32(v):
    return jnp.asarray(np.asarray(v, np.float32))


def _select_token(vband, off, iota_i, iota_f, xor_pats, place_pats,
                  half_masks, grp_pats):
    v = [vband[pl.ds(off + L * j, L)] for j in range(4)]

    gm = list(v)
    for pat in xor_pats:
        gm = [jnp.maximum(x, _vgather(x, pat)) for x in gm]
    placed = [_vgather(gm[j], place_pats[j]) for j in range(4)]
    comb = jnp.where(half_masks[0], placed[0],
                     jnp.where(half_masks[1], placed[1],
                               jnp.where(half_masks[2], placed[2],
                                         placed[3])))
    gq = jnp.where(iota_i < G, comb, NEG)

    gsel = jnp.zeros((L,), jnp.float32)
    for _ in range(KG):
        hmax = jnp.max(gq)
        pos = jnp.min(jnp.where(gq == hmax, iota_i, E))
        gsel = jnp.where(iota_i == pos, 1.0, gsel)
        gq = jnp.where(iota_i == pos, NEG, gq)

    mv = [jnp.where(_vgather(gsel, grp_pats[j]) > 0.0, v[j], NEG)
          for j in range(4)]

    out_idx = jnp.zeros((L,), jnp.int32)
    out_val = jnp.zeros((L,), jnp.float32)
    val0 = jnp.float32(0.0)
    for r in range(K):
        t = jnp.maximum(jnp.maximum(mv[0], mv[1]),
                        jnp.maximum(mv[2], mv[3]))
        hmax = jnp.max(t)
        pos = jnp.min(jnp.where(mv[0] == hmax, iota_i, E))
        for j in range(1, 4):
            pos = jnp.minimum(
                pos, jnp.min(jnp.where(mv[j] == hmax, iota_i + L * j, E)))
        out_idx = jnp.where(iota_i == r, pos, out_idx)
        out_val = jnp.where(iota_i == r, hmax, out_val)
        if r == 0:
            val0 = hmax
        mv = [jnp.where(iota_i == pos - L * j, NEG, mv[j]) for j in range(4)]

    ev = jnp.exp(out_val - val0)
    ev = jnp.where(iota_i < K, ev, 0.0)
    w = ev / jnp.sum(ev)
    return out_idx, w


def _sc_select(logits_flat, T):
    NW = 32
    TPW = T // NW
    mesh = plsc.VectorSubcoreMesh(core_axis_name="c", subcore_axis_name="s")

    @functools.partial(
        pl.kernel,
        out_type=[
            jax.ShapeDtypeStruct((T * K,), jnp.int32),
            jax.ShapeDtypeStruct((T * K,), jnp.float32),
        ],
        mesh=mesh,
        compiler_params=pltpu.CompilerParams(needs_layout_passes=False),
        scratch_types=[
            pltpu.VMEM((TPW * E,), jnp.float32),
            pltpu.VMEM((TPW * K,), jnp.int32),
            pltpu.VMEM((TPW * K,), jnp.float32),
        ],
    )
    def sc_kernel(logits_hbm, idx_hbm, wgt_hbm, band_v, oidx_v, owgt_v):
        iota_i = lax.iota(jnp.int32, L)
        iota_f = iota_i.astype(jnp.float32)
        xor_pats = [jnp.bitwise_xor(iota_i, k) for k in (1, 2, 4)]
        place_pats = [jnp.where(iota_i == 2 * j + 1, 8, 0) for j in range(4)]
        half_masks = [(iota_i >> 1) == j for j in range(3)]
        grp_pats = [(iota_i >> 3) + 2 * j for j in range(4)]
        merge_pat = jnp.maximum(iota_i - 8, 0)
        wid = lax.axis_index("s") * 2 + lax.axis_index("c")
        base = wid * TPW
        pltpu.sync_copy(logits_hbm.at[pl.ds(base * E, TPW * E)], band_v)

        def body(i, carry):
            off = i * (2 * E)
            ia, wa = _select_token(band_v, off, iota_i, iota_f, xor_pats,
                                   place_pats, half_masks, grp_pats)
            ib, wb = _select_token(band_v, off + E, iota_i, iota_f, xor_pats,
                                   place_pats, half_masks, grp_pats)
            midx = jnp.where(iota_i < K, ia, _vgather(ib, merge_pat))
            mwgt = jnp.where(iota_i < K, wa, _vgather(wb, merge_pat))
            oidx_v[pl.ds(i * L, L)] = midx
            owgt_v[pl.ds(i * L, L)] = mwgt
            return carry

        lax.fori_loop(0, TPW // 2, body, 0)
        pltpu.sync_copy(oidx_v, idx_hbm.at[pl.ds(base * K, TPW * K)])
        pltpu.sync_copy(owgt_v, wgt_hbm.at[pl.ds(base * K, TPW * K)])

    return sc_kernel(logits_flat)


def kernel(hidden_states, weight):
    T, D = hidden_states.shape
    logits = _tc_logits(hidden_states, weight)
    idx_flat, wgt_flat = _sc_select(logits.reshape(T * E), T)
    return idx_flat.reshape(T, K), wgt_flat.reshape(T, K)


# speedup vs baseline: 3.9195x; 3.9195x over previous
import jax
import jax.numpy as jnp
from jax.experimental import pallas as pl
from jax.experimental.pallas import tpu as pltpu

E = 64
K = 8
G = 8
KG = 3
GS = E // G


def _gate_kernel(h_ref, w_ref, idx_ref, wgt_ref):
    h = h_ref[...]
    w = w_ref[...]
    logits = jax.lax.dot_general(
        w, h, (((1,), (1,)), ((), ())),
        preferred_element_type=jnp.float32)
    B = logits.shape[1]

    m = jnp.max(logits, axis=0, keepdims=True)
    e = jnp.exp(logits - m)

    ge = jnp.max(e.reshape(G, GS, B), axis=1)

    grows = jax.lax.broadcasted_iota(jnp.int32, ge.shape, 0).astype(jnp.float32)
    gsel = jnp.zeros_like(ge)
    for _ in range(KG):
        gmv = jnp.max(ge, axis=0, keepdims=True)
        gamax = jnp.min(jnp.where(ge == gmv, grows, float(G)),
                        axis=0, keepdims=True)
        hit = grows == gamax
        gsel = jnp.where(hit, 1.0, gsel)
        ge = jnp.where(hit, -1.0, ge)

    onehot = (jax.lax.broadcasted_iota(jnp.int32, (E, G), 0) // GS ==
              jax.lax.broadcasted_iota(jnp.int32, (E, G), 1)).astype(jnp.float32)
    emask = jax.lax.dot_general(
        onehot, gsel, (((1,), (0,)), ((), ())),
        preferred_element_type=jnp.float32)
    cur = jnp.where(emask == 1.0, e, 0.0)

    rows = jax.lax.broadcasted_iota(jnp.int32, cur.shape, 0).astype(jnp.float32)
    idxs, vals = [], []
    for _ in range(K):
        mv = jnp.max(cur, axis=0, keepdims=True)
        amax = jnp.min(jnp.where(cur == mv, rows, float(E)),
                       axis=0, keepdims=True)
        idxs.append(amax)
        vals.append(mv)
        cur = jnp.where(rows == amax, -1.0, cur)
    vals = jnp.concatenate(vals, axis=0)
    idxs_f = jnp.concatenate(idxs, axis=0)
    denom = jnp.sum(vals, axis=0, keepdims=True)
    wgt = vals / denom
    idx_ref[...] = idxs_f.T.astype(jnp.int32)
    wgt_ref[...] = wgt.T


def kernel(hidden_states, weight):
    T, D = hidden_states.shape
    B = 2048
    grid = (T // B,)
    idx, wgt = pl.pallas_call(
        _gate_kernel,
        grid=grid,
        compiler_params=pltpu.CompilerParams(
            dimension_semantics=("parallel",)),
        in_specs=[
            pl.BlockSpec((B, D), lambda i: (i, 0)),
            pl.BlockSpec((E, D), lambda i: (0, 0)),
        ],
        out_specs=[
            pl.BlockSpec((B, K), lambda i: (i, 0)),
            pl.BlockSpec((B, K), lambda i: (i, 0)),
        ],
        out_shape=[
            jax.ShapeDtypeStruct((T, K), jnp.int32),
            jax.ShapeDtypeStruct((T, K), jnp.float32),
        ],
    )(hidden_states, weight)
    return idx, wgt
